# Initial kernel scaffold; baseline (speedup 1.0000x reference)
import jax
import jax.numpy as jnp
from jax.experimental import pallas as pl


def kernel(inputs, pos_table):
    raise NotImplementedError("write your pallas kernel here")


# speedup vs baseline: 1.9052x; 1.9052x over previous
import jax
import jax.numpy as jnp
from jax.experimental import pallas as pl


_BL = 256


def _add_pos_body(x_ref, p_ref, o_ref):
    o_ref[...] = x_ref[...] + p_ref[...][None, :, :]


def kernel(inputs, pos_table):
    B, L, D = inputs.shape
    grid = (L // _BL,)
    return pl.pallas_call(
        _add_pos_body,
        grid=grid,
        in_specs=[
            pl.BlockSpec((B, _BL, D), lambda i: (0, i, 0)),
            pl.BlockSpec((_BL, D), lambda i: (i, 0)),
        ],
        out_specs=pl.BlockSpec((B, _BL, D), lambda i: (0, i, 0)),
        out_shape=jax.ShapeDtypeStruct((B, L, D), inputs.dtype),
    )(inputs, pos_table)
